# baseline (speedup 1.0000x reference)
import functools
import math

import jax
import jax.numpy as jnp
from jax import lax
from jax.experimental import pallas as pl
from jax.experimental.pallas import tpu as pltpu

B, L, D, H = 4, 8192, 768, 12
DK = 64
U = 50
HU = H * U
BLK = 512
NL = L // BLK
NEG = -1e30

_HI = jax.lax.Precision.HIGHEST
_F32 = jnp.float32
_BF16 = jnp.bfloat16


def _dot(a, b, dims, prec=None):
    return lax.dot_general(a, b, dimension_numbers=(dims, ((), ())),
                           precision=prec, preferred_element_type=_F32)


def _prep_body(xs_ref, wq_ref, wk_ref, bq_ref, bk_ref,
               aext_ref, asum_ref, cext_ref, csum_ref):
    xs = xs_ref[0]
    wk = wk_ref[...]
    ks = _dot(xs, wk, ((1,), (1,)), _HI) + bk_ref[...]
    a_cols, s_cols, c_cols, cs_cols = [], [], [], []
    for h in range(H):
        wq_h = wq_ref[h * DK:(h + 1) * DK, :]
        ks_h = ks[:, h * DK:(h + 1) * DK]
        a_h = _dot(wq_h, ks_h, ((0,), (1,)), _HI)
        c_h = _dot(bq_ref[:, h * DK:(h + 1) * DK], ks_h, ((1,), (1,)), _HI)
        a_pad = jnp.concatenate(
            [a_h, jnp.broadcast_to(a_h[:, :1], (D, DK - U))], axis=1)
        c_pad = jnp.concatenate(
            [c_h, jnp.broadcast_to(c_h[:, :1], (1, DK - U))], axis=1)
        a_cols.append(a_pad)
        c_cols.append(c_pad)
        s_cols.append(jnp.sum(a_h, axis=1, keepdims=True) * (1.0 / U))
        cs_cols.append(jnp.sum(c_h, axis=1, keepdims=True) * (1.0 / U))
    aext_ref[0] = jnp.concatenate(a_cols, axis=1)
    cext_ref[0] = jnp.concatenate(c_cols, axis=1)
    asum = jnp.concatenate(s_cols, axis=1)
    csum = jnp.concatenate(cs_cols, axis=1)
    asum_ref[0] = jnp.concatenate(
        [asum, jnp.zeros((D, 128 - H), _F32)], axis=1)
    csum_ref[0] = jnp.concatenate(
        [csum, jnp.zeros((1, 128 - H), _F32)], axis=1)


def _prep(x_samp, Wq, Wk, bq2, bk2):
    return pl.pallas_call(
        _prep_body,
        grid=(B,),
        in_specs=[
            pl.BlockSpec((1, U, D), lambda b: (b, 0, 0)),
            pl.BlockSpec((D, D), lambda b: (0, 0)),
            pl.BlockSpec((D, D), lambda b: (0, 0)),
            pl.BlockSpec((1, D), lambda b: (0, 0)),
            pl.BlockSpec((1, D), lambda b: (0, 0)),
        ],
        out_specs=[
            pl.BlockSpec((1, D, D), lambda b: (b, 0, 0)),
            pl.BlockSpec((1, D, 128), lambda b: (b, 0, 0)),
            pl.BlockSpec((1, 1, D), lambda b: (b, 0, 0)),
            pl.BlockSpec((1, 1, 128), lambda b: (b, 0, 0)),
        ],
        out_shape=[
            jax.ShapeDtypeStruct((B, D, D), _F32),
            jax.ShapeDtypeStruct((B, D, 128), _F32),
            jax.ShapeDtypeStruct((B, 1, D), _F32),
            jax.ShapeDtypeStruct((B, 1, 128), _F32),
        ],
    )(x_samp, Wq, Wk, bq2, bk2)


def _scores_body(x_ref, aext_ref, asum_ref, cext_ref, csum_ref, m_ref):
    xb = x_ref[0]
    s_ext = _dot(xb, aext_ref[0], ((1,), (0,)), _HI) + cext_ref[0]
    s_sum = _dot(xb, asum_ref[0], ((1,), (0,)), _HI) + csum_ref[0]
    smax = jnp.max(s_ext.reshape(BLK, H, DK), axis=-1)
    m12 = smax - s_sum[:, :H]
    m_ref[0] = jnp.concatenate(
        [m12, jnp.full((BLK, 16 - H), NEG, _F32)], axis=1)


def _scores(x, aext, asum, cext, csum):
    return pl.pallas_call(
        _scores_body,
        grid=(B, NL),
        in_specs=[
            pl.BlockSpec((1, BLK, D), lambda b, j: (b, j, 0)),
            pl.BlockSpec((1, D, D), lambda b, j: (b, 0, 0)),
            pl.BlockSpec((1, D, 128), lambda b, j: (b, 0, 0)),
            pl.BlockSpec((1, 1, D), lambda b, j: (b, 0, 0)),
            pl.BlockSpec((1, 1, 128), lambda b, j: (b, 0, 0)),
        ],
        out_specs=pl.BlockSpec((1, BLK, 16), lambda b, j: (b, j, 0)),
        out_shape=jax.ShapeDtypeStruct((B, L, 16), _F32),
    )(x, aext, asum, cext, csum)


def _topk_body(m_ref, out_ref, cur_ref):
    cur_ref[...] = m_ref[...]
    lane = lax.broadcasted_iota(jnp.int32, (B * H, L), 1)
    lane64 = lax.broadcasted_iota(jnp.int32, (B * H, 64), 1)

    def step(j, out):
        cur = cur_ref[...]
        mx = jnp.max(cur, axis=1, keepdims=True)
        idx = jnp.min(jnp.where(cur == mx, lane, L), axis=1, keepdims=True)
        cur_ref[...] = jnp.where(lane == idx, NEG, cur)
        return jnp.where(lane64 == j, idx, out)

    out_ref[...] = lax.fori_loop(0, U, step, jnp.zeros((B * H, 64), jnp.int32))


def _topk(m_rows):
    return pl.pallas_call(
        _topk_body,
        grid=(1,),
        in_specs=[pl.BlockSpec((B * H, L), lambda i: (0, 0))],
        out_specs=pl.BlockSpec((B * H, 64), lambda i: (0, 0)),
        out_shape=jax.ShapeDtypeStruct((B * H, 64), jnp.int32),
        scratch_shapes=[pltpu.VMEM((B * H, L), _F32)],
    )(m_rows)


def _flash_body(x_ref, xt_ref, wq_ref, wk_ref, wv_ref, wo_ref,
                bq_ref, bk_ref, bv_ref, bo_ref,
                delta_ref, base_ref,
                qbd_ref, acc_ref, ml_ref, vsum_ref):
    j = pl.program_id(1)
    row_g = lax.broadcasted_iota(jnp.int32, (HU, D), 0) // U
    col_g = lax.broadcasted_iota(jnp.int32, (HU, D), 1) // DK
    mask = row_g == col_g

    @pl.when(j == 0)
    def _init():
        xt = xt_ref[0].astype(_BF16)
        qt = _dot(xt, wq_ref[...].astype(_BF16), ((1,), (1,))) + bq_ref[...]
        qbd_ref[...] = jnp.where(mask, qt, 0.0).astype(_BF16)
        acc_ref[...] = jnp.zeros((HU, D), _F32)
        ml_ref[:, 0:1] = jnp.full((HU, 1), NEG, _F32)
        ml_ref[:, 1:2] = jnp.zeros((HU, 1), _F32)
        vsum_ref[...] = jnp.zeros((8, D), _F32)

    xb = x_ref[0].astype(_BF16)
    kb = _dot(xb, wk_ref[...].astype(_BF16), ((1,), (1,))) + bk_ref[...]
    vb = _dot(xb, wv_ref[...].astype(_BF16), ((1,), (1,))) + bv_ref[...]
    s = _dot(qbd_ref[...], kb.astype(_BF16), ((1,), (1,))) * (1.0 / math.sqrt(DK))
    mold = ml_ref[:, 0:1]
    mnew = jnp.maximum(mold, jnp.max(s, axis=1, keepdims=True))
    alpha = jnp.exp(mold - mnew)
    p = jnp.exp(s - mnew)
    ml_ref[:, 1:2] = ml_ref[:, 1:2] * alpha + jnp.sum(p, axis=1, keepdims=True)
    ml_ref[:, 0:1] = mnew
    acc_ref[...] = acc_ref[...] * alpha + _dot(p.astype(_BF16),
                                               vb.astype(_BF16), ((1,), (0,)))
    vsum_ref[0:1, :] = vsum_ref[0:1, :] + jnp.sum(vb, axis=0, keepdims=True)

    @pl.when(j == NL - 1)
    def _fin():
        ctx = acc_ref[...] / ml_ref[:, 1:2]
        meanv = vsum_ref[0:1, :] * (1.0 / L)
        cmat = jnp.where(mask, ctx - meanv, 0.0)
        delta_ref[0] = _dot(cmat, wo_ref[...], ((1,), (1,)), _HI)
        base_ref[0] = _dot(meanv, wo_ref[...], ((1,), (1,)), _HI) + bo_ref[...]


def _flash(x, x_top, Wq, Wk, Wv, Wo, bq2, bk2, bv2, bo2):
    return pl.pallas_call(
        _flash_body,
        grid=(B, NL),
        in_specs=[
            pl.BlockSpec((1, BLK, D), lambda b, j: (b, j, 0)),
            pl.BlockSpec((1, HU, D), lambda b, j: (b, 0, 0)),
            pl.BlockSpec((D, D), lambda b, j: (0, 0)),
            pl.BlockSpec((D, D), lambda b, j: (0, 0)),
            pl.BlockSpec((D, D), lambda b, j: (0, 0)),
            pl.BlockSpec((D, D), lambda b, j: (0, 0)),
            pl.BlockSpec((1, D), lambda b, j: (0, 0)),
            pl.BlockSpec((1, D), lambda b, j: (0, 0)),
            pl.BlockSpec((1, D), lambda b, j: (0, 0)),
            pl.BlockSpec((1, D), lambda b, j: (0, 0)),
        ],
        out_specs=[
            pl.BlockSpec((1, HU, D), lambda b, j: (b, 0, 0)),
            pl.BlockSpec((1, 1, D), lambda b, j: (b, 0, 0)),
        ],
        out_shape=[
            jax.ShapeDtypeStruct((B, HU, D), _F32),
            jax.ShapeDtypeStruct((B, 1, D), _F32),
        ],
        scratch_shapes=[
            pltpu.VMEM((HU, D), _BF16),
            pltpu.VMEM((HU, D), _F32),
            pltpu.VMEM((HU, 128), _F32),
            pltpu.VMEM((8, D), _F32),
        ],
    )(x, x_top, Wq, Wk, Wv, Wo, bq2, bk2, bv2, bo2)


def _asm_body(top_ref, delta_ref, base_ref, out_ref):
    j = pl.program_id(1)
    tv = top_ref[0]
    rows = lax.broadcasted_iota(jnp.int32, (BLK, 1), 0) + j * BLK
    p = (rows == tv).astype(_BF16)
    contrib = _dot(p, delta_ref[0], ((1,), (0,)))
    out_ref[0] = contrib + base_ref[0]


def _asm(top3, delta_bf, base):
    return pl.pallas_call(
        _asm_body,
        grid=(B, NL),
        in_specs=[
            pl.BlockSpec((1, 1, HU), lambda b, j: (b, 0, 0)),
            pl.BlockSpec((1, HU, D), lambda b, j: (b, 0, 0)),
            pl.BlockSpec((1, 1, D), lambda b, j: (b, 0, 0)),
        ],
        out_specs=pl.BlockSpec((1, BLK, D), lambda b, j: (b, j, 0)),
        out_shape=jax.ShapeDtypeStruct((B, L, D), _F32),
    )(top3, delta_bf, base)


def kernel(x, Wq, bq, Wk, bk, Wv, bv, Wo, bo):
    idx_k = jax.random.permutation(jax.random.key(42), L)[:U]
    x_samp = x[:, idx_k, :]
    bq2 = bq.reshape(1, D)
    bk2 = bk.reshape(1, D)
    bv2 = bv.reshape(1, D)
    bo2 = bo.reshape(1, D)

    aext, asum, cext, csum = _prep(x_samp, Wq, Wk, bq2, bk2)
    m_out = _scores(x, aext, asum, cext, csum)
    m_rows = m_out[..., :H].transpose(0, 2, 1).reshape(B * H, L)
    top = _topk(m_rows)
    top50 = top[:, :U].reshape(B, HU)

    flat = (top50 + jnp.arange(B, dtype=jnp.int32)[:, None] * L).reshape(-1)
    x_top = x.reshape(B * L, D)[flat].reshape(B, HU, D)

    delta, base = _flash(x, x_top, Wq, Wk, Wv, Wo, bq2, bk2, bv2, bo2)
    out = _asm(top50.reshape(B, 1, HU), delta.astype(_BF16), base)
    return out


# speedup vs baseline: 1.4761x; 1.4761x over previous
import math

import jax
import jax.numpy as jnp
from jax import lax
from jax.experimental import pallas as pl
from jax.experimental.pallas import tpu as pltpu

B, L, D, H = 4, 8192, 768, 12
DK = 64
U = 50
HD = H * DK
BLK = 1024
NL = L // BLK
ABLK = 512
NA = L // ABLK
NEG = -1e30

_HI = jax.lax.Precision.HIGHEST
_H3 = jax.lax.Precision.HIGHEST
_F32 = jnp.float32
_BF16 = jnp.bfloat16


def _dot(a, b, dims, prec=None):
    return lax.dot_general(a, b, dimension_numbers=(dims, ((), ())),
                           precision=prec, preferred_element_type=_F32)


def _prep_body(xs_ref, wq_ref, wk_ref, bq_ref, bk_ref, at_ref, c_ref):
    xs = xs_ref[0]
    ks = _dot(xs, wk_ref[...], ((1,), (1,)), _H3) + bk_ref[...]
    a_rows, c_cols = [], []
    for h in range(H):
        sl = slice(h * DK, (h + 1) * DK)
        ks_h = ks[:, sl]
        a_h = _dot(ks_h, wq_ref[sl, :], ((1,), (0,)), _H3)
        c_h = _dot(bq_ref[:, sl], ks_h, ((1,), (1,)), _H3)
        a_h = a_h - jnp.mean(a_h, axis=0, keepdims=True)
        c_h = c_h - jnp.mean(c_h, axis=1, keepdims=True)
        a_rows.append(jnp.concatenate(
            [a_h, jnp.broadcast_to(a_h[0:1, :], (DK - U, D))], axis=0))
        c_cols.append(jnp.concatenate(
            [c_h, jnp.broadcast_to(c_h[:, 0:1], (1, DK - U))], axis=1))
    at_ref[0] = jnp.concatenate(a_rows, axis=0)
    c_ref[0] = jnp.concatenate(c_cols, axis=1)


def _prep(x_samp, Wq, Wk, bq2, bk2):
    return pl.pallas_call(
        _prep_body,
        grid=(B,),
        in_specs=[
            pl.BlockSpec((1, U, D), lambda b: (b, 0, 0)),
            pl.BlockSpec((D, D), lambda b: (0, 0)),
            pl.BlockSpec((D, D), lambda b: (0, 0)),
            pl.BlockSpec((1, D), lambda b: (0, 0)),
            pl.BlockSpec((1, D), lambda b: (0, 0)),
        ],
        out_specs=[
            pl.BlockSpec((1, HD, D), lambda b: (b, 0, 0)),
            pl.BlockSpec((1, 1, HD), lambda b: (b, 0, 0)),
        ],
        out_shape=[
            jax.ShapeDtypeStruct((B, HD, D), _F32),
            jax.ShapeDtypeStruct((B, 1, HD), _F32),
        ],
    )(x_samp, Wq, Wk, bq2, bk2)


def _scores_body(x_ref, at_ref, c_ref, m_ref):
    xb = x_ref[0]
    s = _dot(xb, at_ref[0], ((1,), (1,)), _H3) + c_ref[0]
    smax = jnp.max(s.reshape(BLK, H, DK), axis=-1)
    m_ref[0] = jnp.concatenate(
        [smax, jnp.full((BLK, 16 - H), NEG, _F32)], axis=1)


def _scores(x, at, c):
    return pl.pallas_call(
        _scores_body,
        grid=(B, NL),
        in_specs=[
            pl.BlockSpec((1, BLK, D), lambda b, j: (b, j, 0)),
            pl.BlockSpec((1, HD, D), lambda b, j: (b, 0, 0)),
            pl.BlockSpec((1, 1, HD), lambda b, j: (b, 0, 0)),
        ],
        out_specs=pl.BlockSpec((1, BLK, 16), lambda b, j: (b, j, 0)),
        out_shape=jax.ShapeDtypeStruct((B, L, 16), _F32),
    )(x, at, c)


def _topk_body(m_ref, out_ref, cur_ref):
    cur_ref[...] = m_ref[...]
    lane = lax.broadcasted_iota(jnp.int32, (B * H, L), 1)
    lane64 = lax.broadcasted_iota(jnp.int32, (B * H, 64), 1)

    def step(j, out):
        cur = cur_ref[...]
        mx = jnp.max(cur, axis=1, keepdims=True)
        idx = jnp.min(jnp.where(cur == mx, lane, L), axis=1, keepdims=True)
        cur_ref[...] = jnp.where(lane == idx, NEG, cur)
        return jnp.where(lane64 == j, idx, out)

    out_ref[...] = lax.fori_loop(
        0, U, step, jnp.full((B * H, 64), -1, jnp.int32))


def _topk(m_rows):
    return pl.pallas_call(
        _topk_body,
        grid=(1,),
        in_specs=[pl.BlockSpec((B * H, L), lambda i: (0, 0))],
        out_specs=pl.BlockSpec((B * H, 64), lambda i: (0, 0)),
        out_shape=jax.ShapeDtypeStruct((B * H, 64), jnp.int32),
        scratch_shapes=[pltpu.VMEM((B * H, L), _F32)],
    )(m_rows)


def _flash_body(x_ref, xt_ref, wq_ref, wk_ref, wv_ref, wo_ref,
                bq_ref, bk_ref, bv_ref, bo_ref,
                delta_ref, base_ref,
                qt_ref, acc_ref, ml_ref, vsum_ref):
    j = pl.program_id(1)

    @pl.when(j == 0)
    def _init():
        xt = xt_ref[0].astype(_BF16)
        wq = wq_ref[...].astype(_BF16)
        for h in range(H):
            sl = slice(h * DK, (h + 1) * DK)
            q_h = _dot(xt[sl, :], wq[sl, :], ((1,), (1,))) + bq_ref[:, sl]
            qt_ref[sl, :] = q_h.astype(_BF16)
        acc_ref[...] = jnp.zeros((HD, DK), _F32)
        ml_ref[:, 0:1] = jnp.full((HD, 1), NEG, _F32)
        ml_ref[:, 1:2] = jnp.zeros((HD, 1), _F32)
        vsum_ref[...] = jnp.zeros((8, D), _F32)

    xb = x_ref[0].astype(_BF16)
    kb = (_dot(xb, wk_ref[...].astype(_BF16), ((1,), (1,)))
          + bk_ref[...]).astype(_BF16)
    vb = (_dot(xb, wv_ref[...].astype(_BF16), ((1,), (1,)))
          + bv_ref[...])
    vb16 = vb.astype(_BF16)
    qt = qt_ref[...]
    s_rows = []
    for h in range(H):
        sl = slice(h * DK, (h + 1) * DK)
        s_rows.append(_dot(qt[sl, :], kb[:, sl], ((1,), (1,))))
    s = jnp.concatenate(s_rows, axis=0) * (1.0 / math.sqrt(DK))
    mold = ml_ref[:, 0:1]
    mnew = jnp.maximum(mold, jnp.max(s, axis=1, keepdims=True))
    alpha = jnp.exp(mold - mnew)
    p = jnp.exp(s - mnew)
    ml_ref[:, 1:2] = ml_ref[:, 1:2] * alpha + jnp.sum(p, axis=1, keepdims=True)
    ml_ref[:, 0:1] = mnew
    p16 = p.astype(_BF16)
    pv_rows = []
    for h in range(H):
        sl = slice(h * DK, (h + 1) * DK)
        pv_rows.append(_dot(p16[sl, :], vb16[:, sl], ((1,), (0,))))
    pv = jnp.concatenate(pv_rows, axis=0)
    acc_ref[...] = acc_ref[...] * alpha + pv
    vsum_ref[0:1, :] = vsum_ref[0:1, :] + jnp.sum(vb, axis=0, keepdims=True)

    @pl.when(j == NL - 1)
    def _fin():
        meanv = vsum_ref[0:1, :] * (1.0 / L)
        ctx = acc_ref[...] / ml_ref[:, 1:2]
        for h in range(H):
            sl = slice(h * DK, (h + 1) * DK)
            ch = ctx[sl, :] - meanv[:, sl]
            delta_ref[0, sl, :] = _dot(ch, wo_ref[:, sl], ((1,), (1,)), _H3)
        base_ref[0] = _dot(meanv, wo_ref[...], ((1,), (1,)), _HI) + bo_ref[...]


def _flash(x, x_top, Wq, Wk, Wv, Wo, bq2, bk2, bv2, bo2):
    return pl.pallas_call(
        _flash_body,
        grid=(B, NL),
        in_specs=[
            pl.BlockSpec((1, BLK, D), lambda b, j: (b, j, 0)),
            pl.BlockSpec((1, HD, D), lambda b, j: (b, 0, 0)),
            pl.BlockSpec((D, D), lambda b, j: (0, 0)),
            pl.BlockSpec((D, D), lambda b, j: (0, 0)),
            pl.BlockSpec((D, D), lambda b, j: (0, 0)),
            pl.BlockSpec((D, D), lambda b, j: (0, 0)),
            pl.BlockSpec((1, D), lambda b, j: (0, 0)),
            pl.BlockSpec((1, D), lambda b, j: (0, 0)),
            pl.BlockSpec((1, D), lambda b, j: (0, 0)),
            pl.BlockSpec((1, D), lambda b, j: (0, 0)),
        ],
        out_specs=[
            pl.BlockSpec((1, HD, D), lambda b, j: (b, 0, 0)),
            pl.BlockSpec((1, 1, D), lambda b, j: (b, 0, 0)),
        ],
        out_shape=[
            jax.ShapeDtypeStruct((B, HD, D), _F32),
            jax.ShapeDtypeStruct((B, 1, D), _F32),
        ],
        scratch_shapes=[
            pltpu.VMEM((HD, DK), _BF16),
            pltpu.VMEM((HD, DK), _F32),
            pltpu.VMEM((HD, 128), _F32),
            pltpu.VMEM((8, D), _F32),
        ],
    )(x, x_top, Wq, Wk, Wv, Wo, bq2, bk2, bv2, bo2)


def _asm_body(top_ref, delta_ref, base_ref, out_ref):
    j = pl.program_id(1)
    tv = top_ref[0]
    rows = lax.broadcasted_iota(jnp.int32, (ABLK, 1), 0) + j * ABLK
    p = (rows == tv).astype(_BF16)
    contrib = _dot(p, delta_ref[0], ((1,), (0,)))
    out_ref[0] = contrib + base_ref[0]


def _asm(top3, delta_bf, base):
    return pl.pallas_call(
        _asm_body,
        grid=(B, NA),
        in_specs=[
            pl.BlockSpec((1, 1, HD), lambda b, j: (b, 0, 0)),
            pl.BlockSpec((1, HD, D), lambda b, j: (b, 0, 0)),
            pl.BlockSpec((1, 1, D), lambda b, j: (b, 0, 0)),
        ],
        out_specs=pl.BlockSpec((1, ABLK, D), lambda b, j: (b, j, 0)),
        out_shape=jax.ShapeDtypeStruct((B, L, D), _F32),
    )(top3, delta_bf, base)


def kernel(x, Wq, bq, Wk, bk, Wv, bv, Wo, bo):
    idx_k = jax.random.permutation(jax.random.key(42), L)[:U]
    x_samp = x[:, idx_k, :]
    bq2 = bq.reshape(1, D)
    bk2 = bk.reshape(1, D)
    bv2 = bv.reshape(1, D)
    bo2 = bo.reshape(1, D)

    at, c = _prep(x_samp, Wq, Wk, bq2, bk2)
    m_out = _scores(x, at, c)
    m_rows = m_out[..., :H].transpose(0, 2, 1).reshape(B * H, L)
    top = _topk(m_rows)
    top_b = top.reshape(B, HD)

    flat = (jnp.maximum(top_b, 0)
            + jnp.arange(B, dtype=jnp.int32)[:, None] * L).reshape(-1)
    x_top = x.reshape(B * L, D)[flat].reshape(B, HD, D)

    delta, base = _flash(x, x_top, Wq, Wk, Wv, Wo, bq2, bk2, bv2, bo2)
    out = _asm(top_b.reshape(B, 1, HD), delta.astype(_BF16), base)
    return out
